# baseline (speedup 1.0000x reference)
import functools

import jax
import jax.numpy as jnp
from jax import lax
from jax.experimental import pallas as pl
from jax.experimental.pallas import tpu as pltpu
from jax.experimental.pallas import tpu_sc as plsc

_BATCH = 16384


@functools.partial(jax.jit, static_argnames=())
def _gather_sc(states, V):
    info = plsc.get_sparse_core_info()
    num_cores = 1
    nw = num_cores * info.num_subcores
    b_per_w = _BATCH // nw
    mesh = plsc.VectorSubcoreMesh(
        core_axis_name="c", subcore_axis_name="s", num_cores=num_cores)

    @functools.partial(
        pl.kernel,
        mesh=mesh,
        out_type=jax.ShapeDtypeStruct((_BATCH,), jnp.float32),
        scratch_types=[
            pltpu.VMEM((b_per_w,), jnp.int32),
            pltpu.VMEM((b_per_w,), jnp.float32),
            pltpu.SemaphoreType.DMA,
            pltpu.SemaphoreType.DMA,
        ],
    )
    def body(states_hbm, table_hbm, out_hbm, idx_v, vals_v, s0, s1):
        wid = lax.axis_index("s") * num_cores + lax.axis_index("c")
        base = wid * b_per_w
        h = b_per_w // 2
        pltpu.sync_copy(states_hbm.at[pl.ds(base, h)], idx_v.at[pl.ds(0, h)])
        g0 = pltpu.async_copy(
            table_hbm.at[idx_v.at[pl.ds(0, h)]], vals_v.at[pl.ds(0, h)], s0)
        pltpu.sync_copy(states_hbm.at[pl.ds(base + h, h)], idx_v.at[pl.ds(h, h)])
        g1 = pltpu.async_copy(
            table_hbm.at[idx_v.at[pl.ds(h, h)]], vals_v.at[pl.ds(h, h)], s1)
        g0.wait()
        o0 = pltpu.async_copy(
            vals_v.at[pl.ds(0, h)], out_hbm.at[pl.ds(base, h)], s0)
        g1.wait()
        pltpu.sync_copy(vals_v.at[pl.ds(h, h)], out_hbm.at[pl.ds(base + h, h)])
        o0.wait()

    return body(states, V)


def kernel(states, V):
    return _gather_sc(states.astype(jnp.int32), V)


# speedup vs baseline: 1.0138x; 1.0138x over previous
import functools

import jax
import jax.numpy as jnp
from jax import lax
from jax.experimental import pallas as pl
from jax.experimental.pallas import tpu as pltpu
from jax.experimental.pallas import tpu_sc as plsc

_BATCH = 16384


@functools.partial(jax.jit, static_argnames=())
def _gather_sc(states, V):
    info = plsc.get_sparse_core_info()
    num_cores = 1
    nw = num_cores * info.num_subcores
    b_per_w = _BATCH // nw
    mesh = plsc.VectorSubcoreMesh(
        core_axis_name="c", subcore_axis_name="s", num_cores=num_cores)

    @functools.partial(
        pl.kernel,
        mesh=mesh,
        out_type=jax.ShapeDtypeStruct((_BATCH,), jnp.float32),
        scratch_types=[
            pltpu.VMEM((b_per_w,), jnp.int32),
            pltpu.VMEM((b_per_w,), jnp.float32),
            pltpu.SemaphoreType.DMA,
            pltpu.SemaphoreType.DMA,
        ],
    )
    def body(states_hbm, table_hbm, out_hbm, idx_v, vals_v, s0, s1):
        wid = lax.axis_index("s") * num_cores + lax.axis_index("c")
        base = wid * b_per_w
        h = b_per_w // 2
        i0 = pltpu.async_copy(
            states_hbm.at[pl.ds(base, h)], idx_v.at[pl.ds(0, h)], s0)
        i1 = pltpu.async_copy(
            states_hbm.at[pl.ds(base + h, h)], idx_v.at[pl.ds(h, h)], s1)
        i0.wait()
        g0 = pltpu.async_copy(
            table_hbm.at[idx_v.at[pl.ds(0, h)]], vals_v.at[pl.ds(0, h)], s0)
        i1.wait()
        g1 = pltpu.async_copy(
            table_hbm.at[idx_v.at[pl.ds(h, h)]], vals_v.at[pl.ds(h, h)], s1)
        g0.wait()
        o0 = pltpu.async_copy(
            vals_v.at[pl.ds(0, h)], out_hbm.at[pl.ds(base, h)], s0)
        g1.wait()
        pltpu.sync_copy(vals_v.at[pl.ds(h, h)], out_hbm.at[pl.ds(base + h, h)])
        o0.wait()

    return body(states, V)


def kernel(states, V):
    return _gather_sc(states.astype(jnp.int32), V)
